# Initial kernel scaffold; baseline (speedup 1.0000x reference)
import jax
import jax.numpy as jnp
from jax.experimental import pallas as pl


def kernel(x, w_in, b_in, bn_in_gamma, bn_in_beta, w_out, b_out, bn_out_gamma, bn_out_beta, embeddings):
    raise NotImplementedError("write your pallas kernel here")


# speedup vs baseline: 1.1827x; 1.1827x over previous
import functools

import jax
import jax.numpy as jnp
from jax import lax
from jax.experimental import pallas as pl
from jax.experimental.pallas import tpu as pltpu
from jax.experimental.pallas import tpu_sc as plsc

B, D, T = 16, 256, 1024
P = 64
PW = 128
K = 1024
N = B * T
EPS = 1e-5
BN_EPS = 1e-5

TT = 512
NTT = T // TT
G = B * NTT
NW = 32
RPW = N // NW
CHUNK = 128
NCH = RPW // CHUNK

_F32 = jnp.float32
_HI = lax.Precision.HIGHEST


def _dot(a, b, dims, precision=None):
    return lax.dot_general(a, b, (dims, ((), ())), precision=precision,
                           preferred_element_type=_F32)


def _k1_body(x_ref, w_ref, b_ref, hraw_ref, bn2_ref, acc_ref):
    i = pl.program_id(0)
    xt = x_ref[0]
    hr = _dot(w_ref[...], xt, ((1,), (0,))) + b_ref[...]
    hraw_ref[0] = hr
    s = jnp.sum(hr, axis=1, keepdims=True)
    s2 = jnp.sum(hr * hr, axis=1, keepdims=True)
    st = jnp.concatenate([s, s2], axis=1)

    @pl.when(i == 0)
    def _():
        acc_ref[...] = st

    @pl.when(i != 0)
    def _():
        acc_ref[...] += st

    @pl.when(i == G - 1)
    def _():
        bn2_ref[...] = acc_ref[...]


def _run_k1(x, w_in, b_in2):
    return pl.pallas_call(
        _k1_body,
        grid=(G,),
        in_specs=[
            pl.BlockSpec((1, D, TT), lambda i: (i // NTT, 0, i % NTT)),
            pl.BlockSpec((P, D), lambda i: (0, 0)),
            pl.BlockSpec((P, 1), lambda i: (0, 0)),
        ],
        out_specs=[
            pl.BlockSpec((1, P, TT), lambda i: (i // NTT, 0, i % NTT)),
            pl.BlockSpec((P, 2), lambda i: (0, 0)),
        ],
        out_shape=[
            jax.ShapeDtypeStruct((B, P, T), _F32),
            jax.ShapeDtypeStruct((P, 2), _F32),
        ],
        scratch_shapes=[pltpu.VMEM((P, 2), _F32)],
    )(x, w_in, b_in2)


def _k2_body(hraw_ref, bn2_ref, gin_ref, betin_ref, emb_ref,
             xperm_ref, hflat_ref, idx_ref, counts_ref, qt_ref,
             mean_s, rstd_s, se_s, cacc_s):
    i = pl.program_id(0)

    @pl.when(i == 0)
    def _():
        m = bn2_ref[:, 0:1] * (1.0 / N)
        m2 = bn2_ref[:, 1:2] * (1.0 / N)
        var = m2 - m * m
        mean_s[...] = m
        rstd_s[...] = jnp.sqrt(var + BN_EPS)
        e = emb_ref[...]
        se_s[...] = jnp.sum(e * e, axis=1, keepdims=True)
        cacc_s[...] = jnp.zeros((K, 1), _F32)

    hr = hraw_ref[0]
    h = (hr - mean_s[...]) / rstd_s[...] * gin_ref[...] + betin_ref[...]
    xperm_ref[0] = h
    hflat_ref[...] = jnp.concatenate(
        [h.T, jnp.zeros((TT, PW - P), _F32)], axis=1)

    sx = jnp.sum(h * h, axis=0, keepdims=True)
    prod = _dot(emb_ref[...], h, ((1,), (0,)))
    dist = (sx + se_s[...]) - 2.0 * prod
    minv = jnp.min(dist, axis=0, keepdims=True)
    iot = lax.broadcasted_iota(jnp.int32, (K, TT), 0)
    sel = jnp.where(dist == minv, iot, K)
    idxv = jnp.min(sel, axis=0, keepdims=True)
    idx_ref[0] = idxv

    oh = (iot == idxv).astype(_F32)
    cacc_s[...] += jnp.sum(oh, axis=1, keepdims=True)
    qt_ref[0] = _dot(emb_ref[...], oh, ((0,), (0,)), _HI)

    @pl.when(i == G - 1)
    def _():
        counts_ref[...] = cacc_s[...]


def _run_k2(hraw, bn2, gin2, betin2, emb):
    return pl.pallas_call(
        _k2_body,
        grid=(G,),
        in_specs=[
            pl.BlockSpec((1, P, TT), lambda i: (i // NTT, 0, i % NTT)),
            pl.BlockSpec((P, 2), lambda i: (0, 0)),
            pl.BlockSpec((P, 1), lambda i: (0, 0)),
            pl.BlockSpec((P, 1), lambda i: (0, 0)),
            pl.BlockSpec((K, P), lambda i: (0, 0)),
        ],
        out_specs=[
            pl.BlockSpec((1, P, TT), lambda i: (i // NTT, 0, i % NTT)),
            pl.BlockSpec((TT, PW), lambda i: (i, 0)),
            pl.BlockSpec((1, 1, TT), lambda i: (i, 0, 0)),
            pl.BlockSpec((K, 1), lambda i: (0, 0)),
            pl.BlockSpec((1, P, TT), lambda i: (i // NTT, 0, i % NTT)),
        ],
        out_shape=[
            jax.ShapeDtypeStruct((B, P, T), _F32),
            jax.ShapeDtypeStruct((N, PW), _F32),
            jax.ShapeDtypeStruct((G, 1, TT), jnp.int32),
            jax.ShapeDtypeStruct((K, 1), _F32),
            jax.ShapeDtypeStruct((B, P, T), _F32),
        ],
        scratch_shapes=[
            pltpu.VMEM((P, 1), _F32), pltpu.VMEM((P, 1), _F32),
            pltpu.VMEM((K, 1), _F32), pltpu.VMEM((K, 1), _F32),
        ],
    )(hraw, bn2, gin2, betin2, emb)


def _k3_body(idx_hbm, hflat_hbm, z_hbm, dw_hbm, idx_v, x_v, dwsh):
    c = lax.axis_index("c")
    s = lax.axis_index("s")
    wid = s * 2 + c

    @pl.when(s == 0)
    def _():
        pltpu.sync_copy(z_hbm, dwsh)
    plsc.subcore_barrier()
    pltpu.sync_copy(hflat_hbm.at[pl.ds(wid * RPW, RPW)], x_v)
    for j in range(NCH):
        pltpu.sync_copy(idx_hbm.at[wid, j], idx_v)
        pltpu.sync_copy(x_v.at[pl.ds(j * CHUNK, CHUNK)],
                        dwsh.at[idx_v], add=True)
    plsc.subcore_barrier()

    @pl.when(s == 0)
    def _():
        pltpu.sync_copy(dwsh, dw_hbm.at[c])


def _make_k3():
    mesh = plsc.VectorSubcoreMesh(core_axis_name="c", subcore_axis_name="s")
    return functools.partial(
        pl.kernel,
        mesh=mesh,
        out_type=jax.ShapeDtypeStruct((2, K, PW), _F32),
        scratch_types=[
            pltpu.VMEM((CHUNK,), jnp.int32),
            pltpu.VMEM((RPW, PW), _F32),
            pltpu.VMEM_SHARED((K, PW), _F32),
        ],
    )(_k3_body)


def _k4_body(qt_ref, wout_ref, bout_ref, gout_ref, betout_ref,
             counts_ref, emb_ref, out_ref, mean_s, ivg_s):
    i = pl.program_id(0)

    @pl.when(i == 0)
    def _():
        e = emb_ref[...]
        cnt = counts_ref[...]
        ec = e * cnt
        qsum = jnp.sum(ec, axis=0, keepdims=True)
        q2 = _dot(ec, e, ((0,), (0,)), _HI)
        gq = _dot(wout_ref[...], q2, ((1,), (0,)), _HI)
        es2 = jnp.sum(gq * wout_ref[...], axis=1, keepdims=True) * (1.0 / N)
        s1 = _dot(wout_ref[...], qsum, ((1,), (1,)), _HI) * (1.0 / N)
        bo = bout_ref[...]
        m = s1 + bo
        eo2 = es2 + 2.0 * bo * s1 + bo * bo
        var = eo2 - m * m
        inv = 1.0 / jnp.sqrt(var + BN_EPS)
        mean_s[...] = m
        ivg_s[...] = inv * gout_ref[...]

    q = qt_ref[0]
    o = _dot(wout_ref[...], q, ((1,), (0,))) + bout_ref[...]
    out_ref[0] = (o - mean_s[...]) * ivg_s[...] + betout_ref[...]


def _run_k4(qt, wout, bout2, gout2, betout2, counts, emb):
    return pl.pallas_call(
        _k4_body,
        grid=(G,),
        in_specs=[
            pl.BlockSpec((1, P, TT), lambda i: (i // NTT, 0, i % NTT)),
            pl.BlockSpec((D, P), lambda i: (0, 0)),
            pl.BlockSpec((D, 1), lambda i: (0, 0)),
            pl.BlockSpec((D, 1), lambda i: (0, 0)),
            pl.BlockSpec((D, 1), lambda i: (0, 0)),
            pl.BlockSpec((K, 1), lambda i: (0, 0)),
            pl.BlockSpec((K, P), lambda i: (0, 0)),
        ],
        out_specs=[pl.BlockSpec((1, D, TT), lambda i: (i // NTT, 0, i % NTT))],
        out_shape=[jax.ShapeDtypeStruct((B, D, T), _F32)],
        scratch_shapes=[pltpu.VMEM((D, 1), _F32), pltpu.VMEM((D, 1), _F32)],
    )(qt, wout, bout2, gout2, betout2, counts, emb)


def _k5_body(dwp_ref, counts_ref, ne_ref):
    om = jnp.float32(1.0 - 0.99)
    dw = (dwp_ref[0] + dwp_ref[1])[:, :P]
    cs = counts_ref[...] * om / om
    n = jnp.sum(cs)
    sm = (cs + EPS) / (n + K * EPS) * n
    dwu = dw * om / om
    ne_ref[...] = dwu / sm


def _run_k5(dw_part, counts):
    return pl.pallas_call(
        _k5_body,
        in_specs=[
            pl.BlockSpec((2, K, PW), lambda: (0, 0, 0)),
            pl.BlockSpec((K, 1), lambda: (0, 0)),
        ],
        out_specs=pl.BlockSpec((K, P), lambda: (0, 0)),
        out_shape=jax.ShapeDtypeStruct((K, P), _F32),
    )(dw_part, counts)


def kernel(x, w_in, b_in, bn_in_gamma, bn_in_beta, w_out, b_out,
           bn_out_gamma, bn_out_beta, embeddings):
    b_in2 = b_in.reshape(P, 1)
    gin2 = bn_in_gamma.reshape(P, 1)
    betin2 = bn_in_beta.reshape(P, 1)
    bout2 = b_out.reshape(D, 1)
    gout2 = bn_out_gamma.reshape(D, 1)
    betout2 = bn_out_beta.reshape(D, 1)

    hraw, bn2 = _run_k1(x, w_in, b_in2)
    x_perm, hflat, idx, counts, qt = _run_k2(hraw, bn2, gin2, betin2,
                                             embeddings)
    idx3 = idx.reshape(NW, NCH, CHUNK)
    zeros = jnp.zeros((K, PW), _F32)
    dw_part = _make_k3()(idx3, hflat, zeros)
    out = _run_k4(qt, w_out, bout2, gout2, betout2, counts, embeddings)[0]
    new_emb = _run_k5(dw_part, counts)
    return (out, x_perm, qt, new_emb)


# speedup vs baseline: 1.4206x; 1.2012x over previous
import functools

import jax
import jax.numpy as jnp
from jax import lax
from jax.experimental import pallas as pl
from jax.experimental.pallas import tpu as pltpu
from jax.experimental.pallas import tpu_sc as plsc

B, D, T = 16, 256, 1024
P = 64
PW = 128
K = 1024
N = B * T
EPS = 1e-5
BN_EPS = 1e-5

TT = 512
NTT = T // TT
G = B * NTT
NW = 32
RPW = N // NW
CHUNK = 128
NCH = RPW // CHUNK

_F32 = jnp.float32
_HI = lax.Precision.HIGHEST


def _dot(a, b, dims, precision=None):
    return lax.dot_general(a, b, (dims, ((), ())), precision=precision,
                           preferred_element_type=_F32)


def _k1_body(x_ref, w_ref, b_ref, hraw_ref, bn2_ref, acc_ref):
    i = pl.program_id(0)
    xt = x_ref[0]
    hr = _dot(w_ref[...], xt, ((1,), (0,))) + b_ref[...]
    hraw_ref[0] = hr
    s = jnp.sum(hr, axis=1, keepdims=True)
    s2 = jnp.sum(hr * hr, axis=1, keepdims=True)
    st = jnp.concatenate([s, s2], axis=1)

    @pl.when(i == 0)
    def _():
        acc_ref[...] = st

    @pl.when(i != 0)
    def _():
        acc_ref[...] += st

    @pl.when(i == G - 1)
    def _():
        bn2_ref[...] = acc_ref[...]


def _run_k1(x, w_in, b_in2):
    return pl.pallas_call(
        _k1_body,
        grid=(G,),
        in_specs=[
            pl.BlockSpec((1, D, TT), lambda i: (i // NTT, 0, i % NTT)),
            pl.BlockSpec((P, D), lambda i: (0, 0)),
            pl.BlockSpec((P, 1), lambda i: (0, 0)),
        ],
        out_specs=[
            pl.BlockSpec((1, P, TT), lambda i: (i // NTT, 0, i % NTT)),
            pl.BlockSpec((P, 2), lambda i: (0, 0)),
        ],
        out_shape=[
            jax.ShapeDtypeStruct((B, P, T), _F32),
            jax.ShapeDtypeStruct((P, 2), _F32),
        ],
        scratch_shapes=[pltpu.VMEM((P, 2), _F32)],
    )(x, w_in, b_in2)


def _k2_body(hraw_ref, bn2_ref, gin_ref, betin_ref, emb_ref,
             xperm_ref, hflat_ref, idx_ref, mean_s, rstd_s, se_s):
    i = pl.program_id(0)

    @pl.when(i == 0)
    def _():
        m = bn2_ref[:, 0:1] * (1.0 / N)
        m2 = bn2_ref[:, 1:2] * (1.0 / N)
        var = m2 - m * m
        mean_s[...] = m
        rstd_s[...] = jnp.sqrt(var + BN_EPS)
        e = emb_ref[...]
        se_s[...] = jnp.sum(e * e, axis=1, keepdims=True)

    hr = hraw_ref[0]
    h = (hr - mean_s[...]) / rstd_s[...] * gin_ref[...] + betin_ref[...]
    xperm_ref[0] = h
    hflat_ref[...] = jnp.concatenate(
        [h.T, jnp.ones((TT, 1), _F32),
         jnp.zeros((TT, PW - P - 1), _F32)], axis=1)

    sx = jnp.sum(h * h, axis=0, keepdims=True)
    prod = _dot(emb_ref[...], h, ((1,), (0,)))
    dist = (sx + se_s[...]) - 2.0 * prod
    minv = jnp.min(dist, axis=0, keepdims=True)
    iot = lax.broadcasted_iota(jnp.int32, (K, TT), 0)
    sel = jnp.where(dist == minv, iot, K)
    idx_ref[0] = jnp.min(sel, axis=0, keepdims=True)


def _run_k2(hraw, bn2, gin2, betin2, emb):
    return pl.pallas_call(
        _k2_body,
        grid=(G,),
        in_specs=[
            pl.BlockSpec((1, P, TT), lambda i: (i // NTT, 0, i % NTT)),
            pl.BlockSpec((P, 2), lambda i: (0, 0)),
            pl.BlockSpec((P, 1), lambda i: (0, 0)),
            pl.BlockSpec((P, 1), lambda i: (0, 0)),
            pl.BlockSpec((K, P), lambda i: (0, 0)),
        ],
        out_specs=[
            pl.BlockSpec((1, P, TT), lambda i: (i // NTT, 0, i % NTT)),
            pl.BlockSpec((TT, PW), lambda i: (i, 0)),
            pl.BlockSpec((1, 1, TT), lambda i: (i, 0, 0)),
        ],
        out_shape=[
            jax.ShapeDtypeStruct((B, P, T), _F32),
            jax.ShapeDtypeStruct((N, PW), _F32),
            jax.ShapeDtypeStruct((G, 1, TT), jnp.int32),
        ],
        scratch_shapes=[
            pltpu.VMEM((P, 1), _F32), pltpu.VMEM((P, 1), _F32),
            pltpu.VMEM((K, 1), _F32),
        ],
    )(hraw, bn2, gin2, betin2, emb)


def _k3_body(idx_hbm, hflat_hbm, z_hbm, emb_hbm, dw_hbm, q_hbm,
             idx_v, x_v, dwsh, sem):
    c = lax.axis_index("c")
    s = lax.axis_index("s")
    wid = s * 2 + c

    @pl.when(s == 0)
    def _():
        pltpu.sync_copy(z_hbm, dwsh)
    plsc.subcore_barrier()
    pltpu.sync_copy(hflat_hbm.at[pl.ds(wid * RPW, RPW)], x_v)
    for j in range(NCH):
        pltpu.sync_copy(idx_hbm.at[wid, j], idx_v)
        pltpu.sync_copy(x_v.at[pl.ds(j * CHUNK, CHUNK)],
                        dwsh.at[idx_v], add=True)
    for j in range(NCH):
        pltpu.sync_copy(idx_hbm.at[wid, j], idx_v)
        pltpu.async_copy(emb_hbm.at[idx_v],
                         x_v.at[pl.ds(j * CHUNK, CHUNK)], sem).wait()
    pltpu.sync_copy(x_v, q_hbm.at[pl.ds(wid * RPW, RPW)])
    plsc.subcore_barrier()

    @pl.when(s == 0)
    def _():
        pltpu.sync_copy(dwsh, dw_hbm.at[c])


def _make_k3():
    mesh = plsc.VectorSubcoreMesh(core_axis_name="c", subcore_axis_name="s")
    return functools.partial(
        pl.kernel,
        mesh=mesh,
        out_type=[
            jax.ShapeDtypeStruct((2, K, PW), _F32),
            jax.ShapeDtypeStruct((N, PW), _F32),
        ],
        scratch_types=[
            pltpu.VMEM((CHUNK,), jnp.int32),
            pltpu.VMEM((RPW, PW), _F32),
            pltpu.VMEM_SHARED((K, PW), _F32),
            pltpu.SemaphoreType.DMA,
        ],
    )(_k3_body)


def _k4_body(q_ref, wout_ref, bout_ref, gout_ref, betout_ref,
             dwp_ref, emb_ref, out_ref, qt_ref, mean_s, ivg_s):
    i = pl.program_id(0)

    @pl.when(i == 0)
    def _():
        e = emb_ref[...]
        cnt = dwp_ref[0, :, P:P + 1] + dwp_ref[1, :, P:P + 1]
        ec = e * cnt
        qsum = jnp.sum(ec, axis=0, keepdims=True)
        q2 = _dot(ec, e, ((0,), (0,)), _HI)
        gq = _dot(wout_ref[...], q2, ((1,), (0,)), _HI)
        es2 = jnp.sum(gq * wout_ref[...], axis=1, keepdims=True) * (1.0 / N)
        s1 = _dot(wout_ref[...], qsum, ((1,), (1,)), _HI) * (1.0 / N)
        bo = bout_ref[...]
        m = s1 + bo
        eo2 = es2 + 2.0 * bo * s1 + bo * bo
        var = eo2 - m * m
        inv = 1.0 / jnp.sqrt(var + BN_EPS)
        mean_s[...] = m
        ivg_s[...] = inv * gout_ref[...]

    q = q_ref[...][:, :P]
    o = _dot(wout_ref[...], q, ((1,), (1,))) + bout_ref[...]
    out_ref[0] = (o - mean_s[...]) * ivg_s[...] + betout_ref[...]
    qt_ref[0] = q.T


def _run_k4(q_flat, wout, bout2, gout2, betout2, dw_part, emb):
    return pl.pallas_call(
        _k4_body,
        grid=(G,),
        in_specs=[
            pl.BlockSpec((TT, PW), lambda i: (i, 0)),
            pl.BlockSpec((D, P), lambda i: (0, 0)),
            pl.BlockSpec((D, 1), lambda i: (0, 0)),
            pl.BlockSpec((D, 1), lambda i: (0, 0)),
            pl.BlockSpec((D, 1), lambda i: (0, 0)),
            pl.BlockSpec((2, K, PW), lambda i: (0, 0, 0)),
            pl.BlockSpec((K, P), lambda i: (0, 0)),
        ],
        out_specs=[
            pl.BlockSpec((1, D, TT), lambda i: (i // NTT, 0, i % NTT)),
            pl.BlockSpec((1, P, TT), lambda i: (i // NTT, 0, i % NTT)),
        ],
        out_shape=[
            jax.ShapeDtypeStruct((B, D, T), _F32),
            jax.ShapeDtypeStruct((B, P, T), _F32),
        ],
        scratch_shapes=[pltpu.VMEM((D, 1), _F32), pltpu.VMEM((D, 1), _F32)],
    )(q_flat, wout, bout2, gout2, betout2, dw_part, emb)


def _k5_body(dwp_ref, ne_ref):
    om = jnp.float32(1.0 - 0.99)
    tot = dwp_ref[0] + dwp_ref[1]
    dw = tot[:, :P]
    cs = tot[:, P:P + 1]
    cs = cs * om / om
    n = jnp.sum(cs)
    sm = (cs + EPS) / (n + K * EPS) * n
    dwu = dw * om / om
    ne_ref[...] = dwu / sm


def _run_k5(dw_part):
    return pl.pallas_call(
        _k5_body,
        in_specs=[pl.BlockSpec((2, K, PW), lambda: (0, 0, 0))],
        out_specs=pl.BlockSpec((K, P), lambda: (0, 0)),
        out_shape=jax.ShapeDtypeStruct((K, P), _F32),
    )(dw_part)


def kernel(x, w_in, b_in, bn_in_gamma, bn_in_beta, w_out, b_out,
           bn_out_gamma, bn_out_beta, embeddings):
    b_in2 = b_in.reshape(P, 1)
    gin2 = bn_in_gamma.reshape(P, 1)
    betin2 = bn_in_beta.reshape(P, 1)
    bout2 = b_out.reshape(D, 1)
    gout2 = bn_out_gamma.reshape(D, 1)
    betout2 = bn_out_beta.reshape(D, 1)

    hraw, bn2 = _run_k1(x, w_in, b_in2)
    x_perm, hflat, idx = _run_k2(hraw, bn2, gin2, betin2, embeddings)
    idx3 = idx.reshape(NW, NCH, CHUNK)
    zeros = jnp.zeros((K, PW), _F32)
    emb128 = jnp.concatenate([embeddings, jnp.zeros((K, PW - P), _F32)], axis=1)
    dw_part, q_flat = _make_k3()(idx3, hflat, zeros, emb128)
    out, qt = _run_k4(q_flat, w_out, bout2, gout2, betout2, dw_part,
                      embeddings)
    new_emb = _run_k5(dw_part)
    return (out, x_perm, qt, new_emb)


# speedup vs baseline: 1.6604x; 1.1688x over previous
import functools

import jax
import jax.numpy as jnp
from jax import lax
from jax.experimental import pallas as pl
from jax.experimental.pallas import tpu as pltpu
from jax.experimental.pallas import tpu_sc as plsc

B, D, T = 16, 256, 1024
P = 64
PW = 128
K = 1024
N = B * T
EPS = 1e-5
BN_EPS = 1e-5

TT = 512
NTT = T // TT
G = B * NTT
NW = 32
RPW = N // NW
CHUNK = 128
NCH = RPW // CHUNK

_F32 = jnp.float32
_HI = lax.Precision.HIGHEST


def _dot(a, b, dims, precision=None):
    return lax.dot_general(a, b, (dims, ((), ())), precision=precision,
                           preferred_element_type=_F32)


def _k1_body(x_ref, w_ref, b_ref, hraw_ref, bn2_ref, acc_ref):
    i = pl.program_id(0)
    xt = x_ref[0]
    hr = _dot(w_ref[...], xt, ((1,), (0,))) + b_ref[...]
    hraw_ref[0] = hr
    s = jnp.sum(hr, axis=1, keepdims=True)
    s2 = jnp.sum(hr * hr, axis=1, keepdims=True)
    st = jnp.concatenate([s, s2], axis=1)

    @pl.when(i == 0)
    def _():
        acc_ref[...] = st

    @pl.when(i != 0)
    def _():
        acc_ref[...] += st

    @pl.when(i == B - 1)
    def _():
        bn2_ref[...] = acc_ref[...]


def _run_k1(x, w_in, b_in2):
    return pl.pallas_call(
        _k1_body,
        grid=(B,),
        in_specs=[
            pl.BlockSpec((1, D, T), lambda i: (i, 0, 0)),
            pl.BlockSpec((P, D), lambda i: (0, 0)),
            pl.BlockSpec((P, 1), lambda i: (0, 0)),
        ],
        out_specs=[
            pl.BlockSpec((1, P, T), lambda i: (i, 0, 0)),
            pl.BlockSpec((P, 2), lambda i: (0, 0)),
        ],
        out_shape=[
            jax.ShapeDtypeStruct((B, P, T), _F32),
            jax.ShapeDtypeStruct((P, 2), _F32),
        ],
        scratch_shapes=[pltpu.VMEM((P, 2), _F32)],
    )(x, w_in, b_in2)


def _k2_body(hraw_ref, bn2_ref, gin_ref, betin_ref, emb_ref,
             xperm_ref, hflat_ref, idx_ref, mean_s, rstd_s, se_s):
    i = pl.program_id(0)

    @pl.when(i == 0)
    def _():
        m = bn2_ref[:, 0:1] * (1.0 / N)
        m2 = bn2_ref[:, 1:2] * (1.0 / N)
        var = m2 - m * m
        mean_s[...] = m
        rstd_s[...] = jnp.sqrt(var + BN_EPS)
        e = emb_ref[...]
        se_s[...] = jnp.sum(e * e, axis=1, keepdims=True)

    hr = hraw_ref[0]
    h = (hr - mean_s[...]) / rstd_s[...] * gin_ref[...] + betin_ref[...]
    xperm_ref[0] = h
    hflat_ref[...] = jnp.concatenate(
        [h.T, jnp.ones((TT, 1), _F32),
         jnp.zeros((TT, PW - P - 1), _F32)], axis=1)

    sx = jnp.sum(h * h, axis=0, keepdims=True)
    prod = _dot(emb_ref[...], h, ((1,), (0,)))
    dist = (sx + se_s[...]) - 2.0 * prod
    minv = jnp.min(dist, axis=0, keepdims=True)
    iot = lax.broadcasted_iota(jnp.int32, (K, TT), 0)
    sel = jnp.where(dist == minv, iot, K)
    idx_ref[0] = jnp.min(sel, axis=0, keepdims=True)


def _run_k2(hraw, bn2, gin2, betin2, emb):
    return pl.pallas_call(
        _k2_body,
        grid=(G,),
        in_specs=[
            pl.BlockSpec((1, P, TT), lambda i: (i // NTT, 0, i % NTT)),
            pl.BlockSpec((P, 2), lambda i: (0, 0)),
            pl.BlockSpec((P, 1), lambda i: (0, 0)),
            pl.BlockSpec((P, 1), lambda i: (0, 0)),
            pl.BlockSpec((K, P), lambda i: (0, 0)),
        ],
        out_specs=[
            pl.BlockSpec((1, P, TT), lambda i: (i // NTT, 0, i % NTT)),
            pl.BlockSpec((TT, PW), lambda i: (i, 0)),
            pl.BlockSpec((1, 1, TT), lambda i: (i, 0, 0)),
        ],
        out_shape=[
            jax.ShapeDtypeStruct((B, P, T), _F32),
            jax.ShapeDtypeStruct((N, PW), _F32),
            jax.ShapeDtypeStruct((G, 1, TT), jnp.int32),
        ],
        scratch_shapes=[
            pltpu.VMEM((P, 1), _F32), pltpu.VMEM((P, 1), _F32),
            pltpu.VMEM((K, 1), _F32),
        ],
    )(hraw, bn2, gin2, betin2, emb)


def _k3_body(idx_hbm, hflat_hbm, z_hbm, emb_hbm, dw_hbm, q_hbm,
             idx_v, x_v, dwsh, sem):
    c = lax.axis_index("c")
    s = lax.axis_index("s")
    wid = s * 2 + c

    @pl.when(s == 0)
    def _():
        pltpu.sync_copy(z_hbm, dwsh)
    plsc.subcore_barrier()
    pltpu.sync_copy(hflat_hbm.at[pl.ds(wid * RPW, RPW)], x_v)
    for j in range(NCH):
        pltpu.sync_copy(idx_hbm.at[wid, j], idx_v)
        pltpu.sync_copy(x_v.at[pl.ds(j * CHUNK, CHUNK)],
                        dwsh.at[idx_v], add=True)
    for j in range(NCH):
        pltpu.sync_copy(idx_hbm.at[wid, j], idx_v)
        pltpu.async_copy(emb_hbm.at[idx_v],
                         x_v.at[pl.ds(j * CHUNK, CHUNK)], sem).wait()
    pltpu.sync_copy(x_v, q_hbm.at[pl.ds(wid * RPW, RPW)])
    plsc.subcore_barrier()

    @pl.when(s == 0)
    def _():
        pltpu.sync_copy(dwsh, dw_hbm.at[c])


def _make_k3():
    mesh = plsc.VectorSubcoreMesh(core_axis_name="c", subcore_axis_name="s")
    return functools.partial(
        pl.kernel,
        mesh=mesh,
        out_type=[
            jax.ShapeDtypeStruct((2, K, PW), _F32),
            jax.ShapeDtypeStruct((N, PW), _F32),
        ],
        scratch_types=[
            pltpu.VMEM((CHUNK,), jnp.int32),
            pltpu.VMEM((RPW, PW), _F32),
            pltpu.VMEM_SHARED((K, PW), _F32),
            pltpu.SemaphoreType.DMA,
        ],
    )(_k3_body)


def _k4_body(q_ref, wout_ref, bout_ref, gout_ref, betout_ref,
             dwp_ref, emb_ref, out_ref, qt_ref, ne_ref, mean_s, ivg_s):
    i = pl.program_id(0)

    @pl.when(i == 0)
    def _():
        e = emb_ref[...]
        cnt = dwp_ref[0, :, P:P + 1] + dwp_ref[1, :, P:P + 1]
        ec = e * cnt
        qsum = jnp.sum(ec, axis=0, keepdims=True)
        q2 = _dot(ec, e, ((0,), (0,)), _HI)
        gq = _dot(wout_ref[...], q2, ((1,), (0,)), _HI)
        es2 = jnp.sum(gq * wout_ref[...], axis=1, keepdims=True) * (1.0 / N)
        s1 = _dot(wout_ref[...], qsum, ((1,), (1,)), _HI) * (1.0 / N)
        bo = bout_ref[...]
        m = s1 + bo
        eo2 = es2 + 2.0 * bo * s1 + bo * bo
        var = eo2 - m * m
        inv = 1.0 / jnp.sqrt(var + BN_EPS)
        mean_s[...] = m
        ivg_s[...] = inv * gout_ref[...]

    q = q_ref[...][:, :P]
    o = _dot(wout_ref[...], q, ((1,), (1,))) + bout_ref[...]
    out_ref[0] = (o - mean_s[...]) * ivg_s[...] + betout_ref[...]
    qt_ref[0] = q.T

    @pl.when(i == B - 1)
    def _():
        om = jnp.float32(1.0 - 0.99)
        tot = dwp_ref[0] + dwp_ref[1]
        dw = tot[:, :P]
        cs = tot[:, P:P + 1] * om / om
        n = jnp.sum(cs)
        sm = (cs + EPS) / (n + K * EPS) * n
        ne_ref[...] = (dw * om / om) / sm


def _run_k4(q_flat, wout, bout2, gout2, betout2, dw_part, emb):
    return pl.pallas_call(
        _k4_body,
        grid=(B,),
        in_specs=[
            pl.BlockSpec((T, PW), lambda i: (i, 0)),
            pl.BlockSpec((D, P), lambda i: (0, 0)),
            pl.BlockSpec((D, 1), lambda i: (0, 0)),
            pl.BlockSpec((D, 1), lambda i: (0, 0)),
            pl.BlockSpec((D, 1), lambda i: (0, 0)),
            pl.BlockSpec((2, K, PW), lambda i: (0, 0, 0)),
            pl.BlockSpec((K, P), lambda i: (0, 0)),
        ],
        out_specs=[
            pl.BlockSpec((1, D, T), lambda i: (i, 0, 0)),
            pl.BlockSpec((1, P, T), lambda i: (i, 0, 0)),
            pl.BlockSpec((K, P), lambda i: (0, 0)),
        ],
        out_shape=[
            jax.ShapeDtypeStruct((B, D, T), _F32),
            jax.ShapeDtypeStruct((B, P, T), _F32),
            jax.ShapeDtypeStruct((K, P), _F32),
        ],
        scratch_shapes=[pltpu.VMEM((D, 1), _F32), pltpu.VMEM((D, 1), _F32)],
    )(q_flat, wout, bout2, gout2, betout2, dw_part, emb)


def kernel(x, w_in, b_in, bn_in_gamma, bn_in_beta, w_out, b_out,
           bn_out_gamma, bn_out_beta, embeddings):
    b_in2 = b_in.reshape(P, 1)
    gin2 = bn_in_gamma.reshape(P, 1)
    betin2 = bn_in_beta.reshape(P, 1)
    bout2 = b_out.reshape(D, 1)
    gout2 = bn_out_gamma.reshape(D, 1)
    betout2 = bn_out_beta.reshape(D, 1)

    hraw, bn2 = _run_k1(x, w_in, b_in2)
    x_perm, hflat, idx = _run_k2(hraw, bn2, gin2, betin2, embeddings)
    idx3 = idx.reshape(NW, NCH, CHUNK)
    zeros = jnp.zeros((K, PW), _F32)
    emb128 = jnp.concatenate([embeddings, jnp.zeros((K, PW - P), _F32)], axis=1)
    dw_part, q_flat = _make_k3()(idx3, hflat, zeros, emb128)
    out, qt, new_emb = _run_k4(q_flat, w_out, bout2, gout2, betout2,
                               dw_part, embeddings)
    return (out, x_perm, qt, new_emb)


# speedup vs baseline: 1.7176x; 1.0344x over previous
import functools

import jax
import jax.numpy as jnp
from jax import lax
from jax.experimental import pallas as pl
from jax.experimental.pallas import tpu as pltpu
from jax.experimental.pallas import tpu_sc as plsc

B, D, T = 16, 256, 1024
P = 64
PW = 128
K = 1024
N = B * T
EPS = 1e-5
BN_EPS = 1e-5

TT = 512
NTT = T // TT
G = B * NTT
NW = 32
RPW = N // NW
CHUNK = 128
NCH = RPW // CHUNK

_F32 = jnp.float32
_HI = lax.Precision.HIGHEST


def _dot(a, b, dims, precision=None):
    return lax.dot_general(a, b, (dims, ((), ())), precision=precision,
                           preferred_element_type=_F32)


def _k1_body(x_ref, w_ref, b_ref, hraw_ref, bn2_ref, acc_ref):
    i = pl.program_id(0)
    xt = x_ref[0]
    hr = _dot(w_ref[...], xt, ((1,), (0,))) + b_ref[...]
    hraw_ref[0] = hr
    s = jnp.sum(hr, axis=1, keepdims=True)
    s2 = jnp.sum(hr * hr, axis=1, keepdims=True)
    st = jnp.concatenate([s, s2], axis=1)

    @pl.when(i == 0)
    def _():
        acc_ref[...] = st

    @pl.when(i != 0)
    def _():
        acc_ref[...] += st

    @pl.when(i == B - 1)
    def _():
        bn2_ref[...] = acc_ref[...]


def _run_k1(x, w_in, b_in2):
    return pl.pallas_call(
        _k1_body,
        grid=(B,),
        in_specs=[
            pl.BlockSpec((1, D, T), lambda i: (i, 0, 0)),
            pl.BlockSpec((P, D), lambda i: (0, 0)),
            pl.BlockSpec((P, 1), lambda i: (0, 0)),
        ],
        out_specs=[
            pl.BlockSpec((1, P, T), lambda i: (i, 0, 0)),
            pl.BlockSpec((P, 2), lambda i: (0, 0)),
        ],
        out_shape=[
            jax.ShapeDtypeStruct((B, P, T), _F32),
            jax.ShapeDtypeStruct((P, 2), _F32),
        ],
        scratch_shapes=[pltpu.VMEM((P, 2), _F32)],
    )(x, w_in, b_in2)


def _k2_body(hraw_ref, bn2_ref, gin_ref, betin_ref, emb_ref,
             xperm_ref, hflat_ref, idx_ref, mean_s, rstd_s, se_s):
    i = pl.program_id(0)

    @pl.when(i == 0)
    def _():
        m = bn2_ref[:, 0:1] * (1.0 / N)
        m2 = bn2_ref[:, 1:2] * (1.0 / N)
        var = m2 - m * m
        mean_s[...] = m
        rstd_s[...] = jnp.sqrt(var + BN_EPS)
        e = emb_ref[...]
        se_s[...] = jnp.sum(e * e, axis=1, keepdims=True)

    hr = hraw_ref[0]
    h = (hr - mean_s[...]) / rstd_s[...] * gin_ref[...] + betin_ref[...]
    xperm_ref[0] = h
    hflat_ref[...] = jnp.concatenate(
        [h.T, jnp.ones((T, 1), _F32),
         jnp.zeros((T, PW - P - 1), _F32)], axis=1)

    sx = jnp.sum(h * h, axis=0, keepdims=True)
    prod = _dot(emb_ref[...], h, ((1,), (0,)))
    dist = (sx + se_s[...]) - 2.0 * prod
    minv = jnp.min(dist, axis=0, keepdims=True)
    iot = lax.broadcasted_iota(jnp.int32, (K, T), 0)
    sel = jnp.where(dist == minv, iot, K)
    idx_ref[0] = jnp.min(sel, axis=0, keepdims=True)


def _run_k2(hraw, bn2, gin2, betin2, emb):
    return pl.pallas_call(
        _k2_body,
        grid=(B,),
        in_specs=[
            pl.BlockSpec((1, P, T), lambda i: (i, 0, 0)),
            pl.BlockSpec((P, 2), lambda i: (0, 0)),
            pl.BlockSpec((P, 1), lambda i: (0, 0)),
            pl.BlockSpec((P, 1), lambda i: (0, 0)),
            pl.BlockSpec((K, P), lambda i: (0, 0)),
        ],
        out_specs=[
            pl.BlockSpec((1, P, T), lambda i: (i, 0, 0)),
            pl.BlockSpec((T, PW), lambda i: (i, 0)),
            pl.BlockSpec((1, 1, T), lambda i: (i, 0, 0)),
        ],
        out_shape=[
            jax.ShapeDtypeStruct((B, P, T), _F32),
            jax.ShapeDtypeStruct((N, PW), _F32),
            jax.ShapeDtypeStruct((B, 1, T), jnp.int32),
        ],
        scratch_shapes=[
            pltpu.VMEM((P, 1), _F32), pltpu.VMEM((P, 1), _F32),
            pltpu.VMEM((K, 1), _F32),
        ],
    )(hraw, bn2, gin2, betin2, emb)


def _k3_body(idx_hbm, hflat_hbm, z_hbm, emb_hbm, dw_hbm, q_hbm,
             idx_v, x_v, dwsh, sem):
    c = lax.axis_index("c")
    s = lax.axis_index("s")
    wid = s * 2 + c

    @pl.when(s == 0)
    def _():
        pltpu.sync_copy(z_hbm, dwsh)
    plsc.subcore_barrier()
    pltpu.sync_copy(hflat_hbm.at[pl.ds(wid * RPW, RPW)], x_v)
    for j in range(NCH):
        pltpu.sync_copy(idx_hbm.at[wid, j], idx_v)
        pltpu.sync_copy(x_v.at[pl.ds(j * CHUNK, CHUNK)],
                        dwsh.at[idx_v], add=True)
    for j in range(NCH):
        pltpu.sync_copy(idx_hbm.at[wid, j], idx_v)
        pltpu.async_copy(emb_hbm.at[idx_v],
                         x_v.at[pl.ds(j * CHUNK, CHUNK)], sem).wait()
    pltpu.sync_copy(x_v, q_hbm.at[pl.ds(wid * RPW, RPW)])
    plsc.subcore_barrier()

    @pl.when(s == 0)
    def _():
        pltpu.sync_copy(dwsh, dw_hbm.at[c])


def _make_k3():
    mesh = plsc.VectorSubcoreMesh(core_axis_name="c", subcore_axis_name="s")
    return functools.partial(
        pl.kernel,
        mesh=mesh,
        out_type=[
            jax.ShapeDtypeStruct((2, K, PW), _F32),
            jax.ShapeDtypeStruct((N, PW), _F32),
        ],
        scratch_types=[
            pltpu.VMEM((CHUNK,), jnp.int32),
            pltpu.VMEM((RPW, PW), _F32),
            pltpu.VMEM_SHARED((K, PW), _F32),
            pltpu.SemaphoreType.DMA,
        ],
    )(_k3_body)


def _k4_body(q_ref, wout_ref, bout_ref, gout_ref, betout_ref,
             dwp_ref, emb_ref, out_ref, qt_ref, ne_ref, mean_s, ivg_s):
    i = pl.program_id(0)

    @pl.when(i == 0)
    def _():
        e = emb_ref[...]
        cnt = dwp_ref[0, :, P:P + 1] + dwp_ref[1, :, P:P + 1]
        ec = e * cnt
        qsum = jnp.sum(ec, axis=0, keepdims=True)
        q2 = _dot(ec, e, ((0,), (0,)), _HI)
        gq = _dot(wout_ref[...], q2, ((1,), (0,)), _HI)
        es2 = jnp.sum(gq * wout_ref[...], axis=1, keepdims=True) * (1.0 / N)
        s1 = _dot(wout_ref[...], qsum, ((1,), (1,)), _HI) * (1.0 / N)
        bo = bout_ref[...]
        m = s1 + bo
        eo2 = es2 + 2.0 * bo * s1 + bo * bo
        var = eo2 - m * m
        inv = 1.0 / jnp.sqrt(var + BN_EPS)
        mean_s[...] = m
        ivg_s[...] = inv * gout_ref[...]

    q = q_ref[...][:, :P]
    o = _dot(wout_ref[...], q, ((1,), (1,))) + bout_ref[...]
    out_ref[0] = (o - mean_s[...]) * ivg_s[...] + betout_ref[...]
    qt_ref[0] = q.T

    @pl.when(i == B - 1)
    def _():
        om = jnp.float32(1.0 - 0.99)
        tot = dwp_ref[0] + dwp_ref[1]
        dw = tot[:, :P]
        cs = tot[:, P:P + 1] * om / om
        n = jnp.sum(cs)
        sm = (cs + EPS) / (n + K * EPS) * n
        ne_ref[...] = (dw * om / om) / sm


def _run_k4(q_flat, wout, bout2, gout2, betout2, dw_part, emb):
    return pl.pallas_call(
        _k4_body,
        grid=(B,),
        in_specs=[
            pl.BlockSpec((T, PW), lambda i: (i, 0)),
            pl.BlockSpec((D, P), lambda i: (0, 0)),
            pl.BlockSpec((D, 1), lambda i: (0, 0)),
            pl.BlockSpec((D, 1), lambda i: (0, 0)),
            pl.BlockSpec((D, 1), lambda i: (0, 0)),
            pl.BlockSpec((2, K, PW), lambda i: (0, 0, 0)),
            pl.BlockSpec((K, P), lambda i: (0, 0)),
        ],
        out_specs=[
            pl.BlockSpec((1, D, T), lambda i: (i, 0, 0)),
            pl.BlockSpec((1, P, T), lambda i: (i, 0, 0)),
            pl.BlockSpec((K, P), lambda i: (0, 0)),
        ],
        out_shape=[
            jax.ShapeDtypeStruct((B, D, T), _F32),
            jax.ShapeDtypeStruct((B, P, T), _F32),
            jax.ShapeDtypeStruct((K, P), _F32),
        ],
        scratch_shapes=[pltpu.VMEM((D, 1), _F32), pltpu.VMEM((D, 1), _F32)],
    )(q_flat, wout, bout2, gout2, betout2, dw_part, emb)


def kernel(x, w_in, b_in, bn_in_gamma, bn_in_beta, w_out, b_out,
           bn_out_gamma, bn_out_beta, embeddings):
    b_in2 = b_in.reshape(P, 1)
    gin2 = bn_in_gamma.reshape(P, 1)
    betin2 = bn_in_beta.reshape(P, 1)
    bout2 = b_out.reshape(D, 1)
    gout2 = bn_out_gamma.reshape(D, 1)
    betout2 = bn_out_beta.reshape(D, 1)

    hraw, bn2 = _run_k1(x, w_in, b_in2)
    x_perm, hflat, idx = _run_k2(hraw, bn2, gin2, betin2, embeddings)
    idx3 = idx.reshape(NW, NCH, CHUNK)
    zeros = jnp.zeros((K, PW), _F32)
    emb128 = jnp.concatenate([embeddings, jnp.zeros((K, PW - P), _F32)], axis=1)
    dw_part, q_flat = _make_k3()(idx3, hflat, zeros, emb128)
    out, qt, new_emb = _run_k4(q_flat, w_out, bout2, gout2, betout2,
                               dw_part, embeddings)
    return (out, x_perm, qt, new_emb)
